# baseline (speedup 1.0000x reference)
import functools

import jax
import jax.numpy as jnp
from jax import lax
from jax.experimental import pallas as pl
from jax.experimental.pallas import tpu as pltpu
from jax.experimental.pallas import tpu_sc as plsc

HEADS = 16
BUCKETS = 64
SEQ = 4096
DIM = 128
TOK = SEQ // BUCKETS

S_SC = 16
NW = 32
NLANE = 16
NGRP = DIM // NLANE
ROW_UNROLL = 8
CHUNK_B = 4
CHUNK_R = CHUNK_B * TOK
NCHUNK = SEQ // CHUNK_R


def _reduce_bucket(buf, slot, row0):
    def body(it, accs):
        base = row0 + it * ROW_UNROLL
        new = list(accs)
        for i in range(ROW_UNROLL):
            for g in range(NGRP):
                new[g] = new[g] + buf[slot, base + i, pl.ds(g * NLANE, NLANE)]
        return tuple(new)
    init = tuple(jnp.zeros((NLANE,), jnp.float32) for _ in range(NGRP))
    return lax.fori_loop(0, TOK // ROW_UNROLL, body, init)


def _sc_means_body(q_hbm, k_hbm, out_hbm, buf, acc, sem0, sem1):
    wid = lax.axis_index("s") * 2 + lax.axis_index("c")
    sems = (sem0, sem1)

    def process(src, sl):
        pltpu.make_async_copy(
            src.at[sl, pl.ds(0, CHUNK_R)], buf.at[0], sems[0]).start()

        def outer(c2, _):
            for par in range(2):
                c = c2 * 2 + par
                @pl.when(c + 1 < NCHUNK)
                def _start_next():
                    pltpu.make_async_copy(
                        src.at[sl, pl.ds((c + 1) * CHUNK_R, CHUNK_R)],
                        buf.at[1 - par], sems[1 - par]).start()
                pltpu.make_async_copy(
                    src.at[sl, pl.ds(c * CHUNK_R, CHUNK_R)],
                    buf.at[par], sems[par]).wait()
                for b in range(CHUNK_B):
                    sums = _reduce_bucket(buf, par, b * TOK)
                    for g in range(NGRP):
                        acc[c * CHUNK_B + b, pl.ds(g * NLANE, NLANE)] = (
                            sums[g] * jnp.float32(1.0 / TOK))
            return 0

        lax.fori_loop(0, NCHUNK // 2, outer, 0)

    @pl.when(wid < S_SC)
    def _q_side():
        process(q_hbm, wid)
    @pl.when(wid >= S_SC)
    def _k_side():
        process(k_hbm, wid - S_SC)
    pltpu.sync_copy(acc, out_hbm.at[pl.ds(wid * BUCKETS, BUCKETS)])


def _sc_means(q, k):
    mesh = plsc.VectorSubcoreMesh(core_axis_name="c", subcore_axis_name="s")
    f = pl.kernel(
        _sc_means_body,
        out_type=jax.ShapeDtypeStruct((2 * S_SC * BUCKETS, DIM), jnp.float32),
        mesh=mesh,
        scratch_types=[
            pltpu.VMEM((2, CHUNK_R, DIM), jnp.float32),
            pltpu.VMEM((BUCKETS, DIM), jnp.float32),
            pltpu.SemaphoreType.DMA,
            pltpu.SemaphoreType.DMA,
        ],
    )
    return f(q, k)


def _sortnet(mq, mk, qpos, kpos, wq, wk):
    sq = (jnp.dot(mq, wq[:DIM], preferred_element_type=jnp.float32)
          + jnp.dot(qpos, wq[DIM:], preferred_element_type=jnp.float32))
    sk = (jnp.dot(mk, wk[:DIM], preferred_element_type=jnp.float32)
          + jnp.dot(kpos, wk[DIM:], preferred_element_type=jnp.float32))
    r = lax.dot_general(sq, sk, (((1,), (1,)), ((), ())),
                        preferred_element_type=jnp.float32)
    r = r - jnp.max(r, axis=-1, keepdims=True)
    e = jnp.exp(r)
    return e / jnp.sum(e, axis=-1, keepdims=True)


def _tail_body(mq_ref, mk_ref, qpos_ref, kpos_ref, wq_ref, wk_ref, out_ref):
    out_ref[0] = _sortnet(mq_ref[0], mk_ref[0], qpos_ref[0, 0], kpos_ref[0, 0],
                          wq_ref[0, 0], wk_ref[0, 0])


def _tc_tail(means, q_pos_emb, k_pos_emb, linear_sort_q, linear_sort_k):
    mq = means[:S_SC * BUCKETS].reshape(S_SC, BUCKETS, DIM)
    mk = means[S_SC * BUCKETS:].reshape(S_SC, BUCKETS, DIM)
    return pl.pallas_call(
        _tail_body,
        grid=(S_SC,),
        in_specs=[
            pl.BlockSpec((1, BUCKETS, DIM), lambda i: (i, 0, 0)),
            pl.BlockSpec((1, BUCKETS, DIM), lambda i: (i, 0, 0)),
            pl.BlockSpec((1, 1, BUCKETS, DIM), lambda i: (0, i % HEADS, 0, 0)),
            pl.BlockSpec((1, 1, BUCKETS, DIM), lambda i: (0, i % HEADS, 0, 0)),
            pl.BlockSpec((1, 1, 2 * DIM, DIM), lambda i: (0, i % HEADS, 0, 0)),
            pl.BlockSpec((1, 1, 2 * DIM, DIM), lambda i: (0, i % HEADS, 0, 0)),
        ],
        out_specs=pl.BlockSpec((1, BUCKETS, BUCKETS), lambda i: (i, 0, 0)),
        out_shape=jax.ShapeDtypeStruct((S_SC, BUCKETS, BUCKETS), jnp.float32),
    )(mq, mk, q_pos_emb, k_pos_emb, linear_sort_q, linear_sort_k)


def _fused_body(q_ref, k_ref, qpos_ref, kpos_ref, wq_ref, wk_ref, out_ref):
    mq = jnp.sum(q_ref[0].reshape(BUCKETS, TOK, DIM), axis=1) * (
        jnp.float32(1.0 / TOK))
    mk = jnp.sum(k_ref[0].reshape(BUCKETS, TOK, DIM), axis=1) * (
        jnp.float32(1.0 / TOK))
    out_ref[0] = _sortnet(mq, mk, qpos_ref[0, 0], kpos_ref[0, 0],
                          wq_ref[0, 0], wk_ref[0, 0])


def _tc_fused(q, k, q_pos_emb, k_pos_emb, linear_sort_q, linear_sort_k):
    n = q.shape[0] - S_SC
    return pl.pallas_call(
        _fused_body,
        grid=(n,),
        in_specs=[
            pl.BlockSpec((1, SEQ, DIM), lambda i: (i + S_SC, 0, 0)),
            pl.BlockSpec((1, SEQ, DIM), lambda i: (i + S_SC, 0, 0)),
            pl.BlockSpec((1, 1, BUCKETS, DIM),
                         lambda i: (0, (i + S_SC) % HEADS, 0, 0)),
            pl.BlockSpec((1, 1, BUCKETS, DIM),
                         lambda i: (0, (i + S_SC) % HEADS, 0, 0)),
            pl.BlockSpec((1, 1, 2 * DIM, DIM),
                         lambda i: (0, (i + S_SC) % HEADS, 0, 0)),
            pl.BlockSpec((1, 1, 2 * DIM, DIM),
                         lambda i: (0, (i + S_SC) % HEADS, 0, 0)),
        ],
        out_specs=pl.BlockSpec((1, BUCKETS, BUCKETS), lambda i: (i, 0, 0)),
        out_shape=jax.ShapeDtypeStruct((n, BUCKETS, BUCKETS), jnp.float32),
    )(q, k, q_pos_emb, k_pos_emb, linear_sort_q, linear_sort_k)


def kernel(q, k, q_pos_emb, k_pos_emb, linear_sort_q, linear_sort_k):
    means = _sc_means(q, k)
    out_tc = _tc_fused(q, k, q_pos_emb, k_pos_emb, linear_sort_q, linear_sort_k)
    out_sc = _tc_tail(means, q_pos_emb, k_pos_emb, linear_sort_q, linear_sort_k)
    return jnp.concatenate([out_sc, out_tc], axis=0)
